# baseline (speedup 1.0000x reference)
import functools

import jax
import jax.numpy as jnp
from jax import lax
from jax.experimental import pallas as pl

_R = 144


def _rownorm_sumsq(v):
    sq = v * v
    n = sq.shape[1]
    s = sq[:, 0:8]
    for k in range(1, n // 8):
        s = s + sq[:, 8 * k:8 * (k + 1)]
    w = 8
    while w > 1:
        s = s[:, :w // 2] + s[:, w // 2:w]
        w //= 2
    return s


def _merge_body(m_ref, x_ref, o_ref, *, half, r, c, big, nb):
    for bb in range(nb):
        _merge_one(m_ref, x_ref, o_ref, bb, half=half, r=r, c=c, big=big)


def _merge_one(m_ref, x_ref, o_ref, bb, *, half, r, c, big):
    mm = m_ref[bb]
    d = mm.shape[1] // 2
    a = mm[:, :d]
    b = mm[:, d:]
    a = a / jnp.sqrt(_rownorm_sumsq(a))
    b = b / jnp.sqrt(_rownorm_sumsq(b))
    s = lax.dot_general(a, b, (((1,), (1,)), ((), ())),
                        preferred_element_type=jnp.float32)

    nm = jnp.max(s, axis=1, keepdims=True)
    ii = lax.broadcasted_iota(jnp.int32, (half, half), 0)
    jj = lax.broadcasted_iota(jnp.int32, (half, half), 1)
    nidx = jnp.min(jnp.where(s == nm, jj, big), axis=1, keepdims=True)

    nm_row = jnp.swapaxes(nm, 0, 1)

    cmp = ((nm_row > nm) | ((nm_row == nm) & (jj < ii))).astype(jnp.float32)
    rank = lax.dot_general(cmp, jnp.ones((half, 1), jnp.float32),
                           (((1,), (0,)), ((), ())),
                           preferred_element_type=jnp.float32)
    rank = rank.astype(jnp.int32)

    unm = half - r
    o_idx = jnp.where(rank >= r, rank - r, unm + nidx)
    o_row = jnp.swapaxes(o_idx, 0, 1)

    nout = unm + half
    oo = lax.broadcasted_iota(jnp.int32, (nout, half), 0)
    ct = (oo == o_row).astype(jnp.float32)

    xx = x_ref[bb]
    xe = xx[:, :c]
    xo = xx[:, c:]
    osum = lax.dot_general(ct, xe, (((1,), (0,)), ((), ())),
                           preferred_element_type=jnp.float32)
    osum = osum + jnp.concatenate(
        [jnp.zeros((unm, c), jnp.float32), xo], axis=0)

    w = lax.dot_general(ct, jnp.ones((half, 1), jnp.float32),
                        (((1,), (0,)), ((), ())),
                        preferred_element_type=jnp.float32)
    is_dst = (lax.broadcasted_iota(jnp.int32, (nout, 1), 0) >= unm)
    w = w + is_dst.astype(jnp.float32)
    o_ref[bb] = osum / w


def kernel(x, metric):
    bsz, t, c = x.shape
    d = metric.shape[-1]
    half = t // 2
    r = min(_R, half)
    nout = (half - r) + half

    x2 = x.reshape(bsz, half, 2 * c)
    m2 = metric.reshape(bsz, half, 2 * d)

    nb = 4
    body = functools.partial(_merge_body, half=half, r=r, c=c, big=1 << 30,
                             nb=nb)
    out = pl.pallas_call(
        body,
        grid=(bsz // nb,),
        in_specs=[
            pl.BlockSpec((nb, half, 2 * d), lambda i: (i, 0, 0)),
            pl.BlockSpec((nb, half, 2 * c), lambda i: (i, 0, 0)),
        ],
        out_specs=pl.BlockSpec((nb, nout, c), lambda i: (i, 0, 0)),
        out_shape=jax.ShapeDtypeStruct((bsz, nout, c), jnp.float32),
    )(m2, x2)
    return out
